# baseline (speedup 1.0000x reference)
import functools

import jax
import jax.numpy as jnp
from jax import lax
from jax.experimental import pallas as pl
from jax.experimental.pallas import tpu as pltpu
from jax.experimental.pallas import tpu_sc as plsc

GAMMA = 0.5
LANES = 16
GATHER_CHUNK = 128


def _permute(v, idx16):
    dnums = lax.GatherDimensionNumbers(
        offset_dims=(), collapsed_slice_dims=(0,), start_index_map=(0,))
    return lax.gather(v, idx16[:, None], dnums, slice_sizes=(1,),
                      mode=lax.GatherScatterMode.PROMISE_IN_BOUNDS)


@functools.partial(jax.jit, static_argnums=(2, 3))
def _sc_rbf(x, idx, batch, dim):
    info = plsc.get_sparse_core_info()
    num_cores, num_subcores = info.num_cores, info.num_subcores
    num_workers = num_cores * num_subcores
    bpw = batch // num_workers

    mesh = plsc.VectorSubcoreMesh(core_axis_name="c", subcore_axis_name="s")

    @functools.partial(
        pl.kernel,
        mesh=mesh,
        out_type=jax.ShapeDtypeStruct((batch,), jnp.float32),
        compiler_params=pltpu.CompilerParams(use_tc_tiling_on_sc=False),
        scratch_types=[
            pltpu.VMEM((bpw,), jnp.int32),
            pltpu.VMEM((bpw, dim), jnp.float32),
            pltpu.VMEM((bpw, dim), jnp.float32),
            pltpu.VMEM((bpw,), jnp.float32),
            pltpu.SemaphoreType.DMA,
        ],
    )
    def k(x_hbm, idx_hbm, out_hbm, idx_v, x_v, cent_v, sums_v, sem):
        wid = lax.axis_index("s") * num_cores + lax.axis_index("c")
        base = wid * bpw

        pltpu.sync_copy(idx_hbm.at[pl.ds(base, bpw)], idx_v)

        copies = []
        for c in range(bpw // GATHER_CHUNK):
            copies.append(
                pltpu.async_copy(
                    x_hbm.at[idx_v.at[pl.ds(c * GATHER_CHUNK, GATHER_CHUNK)]],
                    cent_v.at[pl.ds(c * GATHER_CHUNK, GATHER_CHUNK), :],
                    sem,
                )
            )
        pltpu.sync_copy(x_hbm.at[pl.ds(base, bpw), :], x_v)
        for cp in copies:
            cp.wait()

        lane = lax.iota(jnp.int32, LANES)
        perms = [lane ^ (1 << k) for k in range(4)]

        def grp(g, carry):
            off = pl.multiple_of(g * LANES, LANES)
            out_vec = jnp.zeros((LANES,), jnp.float32)
            for l in range(LANES):
                r = off + l
                acc = jnp.zeros((LANES,), jnp.float32)
                for c4 in range(dim // LANES):
                    dx = (x_v[r, pl.ds(c4 * LANES, LANES)]
                          - cent_v[r, pl.ds(c4 * LANES, LANES)])
                    acc = acc + dx * dx
                for p in perms:
                    acc = acc + _permute(acc, p)
                out_vec = jnp.where(lane == l, acc, out_vec)
            sums_v[pl.ds(off, LANES)] = jnp.exp(-GAMMA * out_vec)
            return carry

        lax.fori_loop(0, bpw // LANES, grp, 0)

        pltpu.sync_copy(sums_v, out_hbm.at[pl.ds(base, bpw)])

    return k(x, idx)


def kernel(x, rng):
    if x.ndim == 1:
        x = x.reshape(-1, 1)
    batch, dim = x.shape
    centers_idx = jax.random.choice(rng, jnp.arange(batch), shape=(batch,))
    return _sc_rbf(x, centers_idx.astype(jnp.int32), batch, dim)


# speedup vs baseline: 1.1078x; 1.1078x over previous
import functools

import jax
import jax.numpy as jnp
from jax import lax
from jax.experimental import pallas as pl
from jax.experimental.pallas import tpu as pltpu
from jax.experimental.pallas import tpu_sc as plsc

GAMMA = 0.5
LANES = 16
GATHER_CHUNK = 128


def _permute(v, idx16):
    dnums = lax.GatherDimensionNumbers(
        offset_dims=(), collapsed_slice_dims=(0,), start_index_map=(0,))
    return lax.gather(v, idx16[:, None], dnums, slice_sizes=(1,),
                      mode=lax.GatherScatterMode.PROMISE_IN_BOUNDS)


@functools.partial(jax.jit, static_argnums=(2, 3))
def _sc_rbf(x, idx, batch, dim):
    info = plsc.get_sparse_core_info()
    num_cores, num_subcores = info.num_cores, info.num_subcores
    num_workers = num_cores * num_subcores
    bpw = batch // num_workers

    mesh = plsc.VectorSubcoreMesh(core_axis_name="c", subcore_axis_name="s")

    @functools.partial(
        pl.kernel,
        mesh=mesh,
        out_type=jax.ShapeDtypeStruct((batch,), jnp.float32),
        compiler_params=pltpu.CompilerParams(use_tc_tiling_on_sc=False),
        scratch_types=[
            pltpu.VMEM((bpw,), jnp.int32),
            pltpu.VMEM((bpw, dim), jnp.float32),
            pltpu.VMEM((bpw, dim), jnp.float32),
            pltpu.VMEM((bpw,), jnp.float32),
            pltpu.SemaphoreType.DMA,
        ],
    )
    def k(x_hbm, idx_hbm, out_hbm, idx_v, x_v, cent_v, sums_v, sem):
        wid = lax.axis_index("s") * num_cores + lax.axis_index("c")
        base = wid * bpw

        pltpu.sync_copy(idx_hbm.at[pl.ds(base, bpw)], idx_v)

        copies = []
        for c in range(bpw // GATHER_CHUNK):
            copies.append(
                pltpu.async_copy(
                    x_hbm.at[idx_v.at[pl.ds(c * GATHER_CHUNK, GATHER_CHUNK)]],
                    cent_v.at[pl.ds(c * GATHER_CHUNK, GATHER_CHUNK), :],
                    sem,
                )
            )
        pltpu.sync_copy(x_hbm.at[pl.ds(base, bpw), :], x_v)
        for cp in copies:
            cp.wait()

        lane = lax.iota(jnp.int32, LANES)
        perms = [lane ^ (1 << k) for k in range(4)]

        def grp(g, carry):
            off = pl.multiple_of(g * LANES, LANES)
            out_vec = jnp.zeros((LANES,), jnp.float32)
            for l in range(LANES):
                r = off + l
                acc = jnp.zeros((LANES,), jnp.float32)
                for c4 in range(dim // LANES):
                    dx = (x_v[r, pl.ds(c4 * LANES, LANES)]
                          - cent_v[r, pl.ds(c4 * LANES, LANES)])
                    acc = acc + dx * dx
                for p in perms:
                    acc = acc + _permute(acc, p)
                out_vec = jnp.where(lane == l, acc, out_vec)
            sums_v[pl.ds(off, LANES)] = jnp.exp(-GAMMA * out_vec)
            return carry

        lax.fori_loop(0, bpw // LANES, grp, 0)

        pltpu.sync_copy(sums_v, out_hbm.at[pl.ds(base, bpw)])

    return k(x, idx)


def kernel(x, rng):
    if x.ndim == 1:
        x = x.reshape(-1, 1)
    batch, dim = x.shape
    centers_idx = jax.random.randint(rng, (batch,), 0, batch)
    return _sc_rbf(x, centers_idx.astype(jnp.int32), batch, dim)
